# baseline (device time: 28584 ns/iter reference)
import jax
import jax.numpy as jnp
from jax import lax
from jax.experimental import pallas as pl
from jax.experimental.pallas import tpu as pltpu

N_DEV = 32
N_BLK = 4


def _by_far(js):
    return sorted(js, key=lambda j: -min(j, N_DEV - j))


def kernel(table, idx):
    rows_per, d = table.shape
    n = idx.shape[0]
    chunk = n // N_DEV
    blk = n // N_BLK

    def body(table_ref, idx2_ref, out_ref, partial, rs_buf,
             entry_sems, rs_send, rs_recv, ag_send, ag_recv):
        my = lax.axis_index("i")

        for j in range(1, N_DEV):
            pl.semaphore_signal(
                entry_sems.at[j], inc=1,
                device_id=(jnp.mod(my + N_DEV - j, N_DEV),),
                device_id_type=pl.DeviceIdType.MESH,
            )

        barrier = pltpu.get_barrier_semaphore()
        for nbr in (jnp.mod(my + 1, N_DEV), jnp.mod(my + N_DEV - 1, N_DEV)):
            pl.semaphore_signal(
                barrier, inc=1,
                device_id=(nbr,), device_id_type=pl.DeviceIdType.MESH,
            )

        tv = table_ref[...].astype(jnp.bfloat16)
        base = my * chunk
        col = lax.broadcasted_iota(jnp.int32, (blk, rows_per), 1)

        rs = []
        first = True
        for b in range(N_BLK):
            ids = idx2_ref[pl.ds(base + b * blk, blk), :]
            li = ids - my * rows_per
            in_range = (li >= 0) & (li < rows_per)
            oh = ((col == li) & in_range).astype(jnp.bfloat16)
            partial[pl.ds(b * blk, blk), :] = jnp.dot(
                oh, tv, preferred_element_type=jnp.float32
            )

            if first:
                pl.semaphore_wait(barrier, 2)
                first = False

            for j in _by_far(range(max(b * N_DEV // N_BLK, 1),
                                   (b + 1) * N_DEV // N_BLK)):
                pl.semaphore_wait(entry_sems.at[j], 1)
                r = pltpu.make_async_remote_copy(
                    src_ref=partial.at[pl.ds(j * chunk, chunk)],
                    dst_ref=rs_buf.at[j],
                    send_sem=rs_send.at[j],
                    recv_sem=rs_recv.at[j],
                    device_id=(jnp.mod(my + j, N_DEV),),
                    device_id_type=pl.DeviceIdType.MESH,
                )
                r.start()
                rs.append(r)

        for r in rs:
            r.wait_recv()
        rs_buf[0, :, :] = partial[0:chunk, :]
        vals = [rs_buf[j] for j in range(N_DEV)]
        while len(vals) > 1:
            vals = [
                vals[i] + vals[i + 1] if i + 1 < len(vals) else vals[i]
                for i in range(0, len(vals), 2)
            ]
        out_ref[pl.ds(my * chunk, chunk), :] = vals[0]

        ag = []
        for j in _by_far(range(1, N_DEV)):
            r = pltpu.make_async_remote_copy(
                src_ref=out_ref.at[pl.ds(my * chunk, chunk)],
                dst_ref=out_ref.at[pl.ds(my * chunk, chunk)],
                send_sem=ag_send.at[j],
                recv_sem=ag_recv.at[j],
                device_id=(jnp.mod(my + j, N_DEV),),
                device_id_type=pl.DeviceIdType.MESH,
            )
            r.start()
            ag.append(r)

        for j in range(1, N_DEV):
            src_dev = jnp.mod(my + (N_DEV - j), N_DEV)
            w = pltpu.make_async_remote_copy(
                src_ref=out_ref.at[pl.ds(src_dev * chunk, chunk)],
                dst_ref=out_ref.at[pl.ds(src_dev * chunk, chunk)],
                send_sem=ag_send.at[j],
                recv_sem=ag_recv.at[j],
                device_id=(src_dev,),
                device_id_type=pl.DeviceIdType.MESH,
            )
            w.wait_recv()

        for r in rs:
            r.wait_send()
        for r in ag:
            r.wait_send()

    idx2 = jnp.concatenate([idx, idx]).reshape(2 * n, 1)

    return pl.pallas_call(
        body,
        out_shape=jax.ShapeDtypeStruct((n, d), jnp.float32),
        in_specs=[
            pl.BlockSpec(memory_space=pltpu.VMEM),
            pl.BlockSpec(memory_space=pltpu.VMEM),
        ],
        out_specs=pl.BlockSpec(memory_space=pltpu.VMEM),
        scratch_shapes=[
            pltpu.VMEM((n, d), jnp.float32),
            pltpu.VMEM((N_DEV, chunk, d), jnp.float32),
            pltpu.SemaphoreType.REGULAR((N_DEV,)),
            pltpu.SemaphoreType.DMA((N_DEV,)),
            pltpu.SemaphoreType.DMA((N_DEV,)),
            pltpu.SemaphoreType.DMA((N_DEV,)),
            pltpu.SemaphoreType.DMA((N_DEV,)),
        ],
        compiler_params=pltpu.CompilerParams(collective_id=0),
    )(table, idx2)
